# Initial kernel scaffold; baseline (speedup 1.0000x reference)
import jax
import jax.numpy as jnp
from jax.experimental import pallas as pl


def kernel(x, edge_index, positions, W0, b0, ln_g, ln_b, Wq, bq, Wk, bk, Wv, bv, gate):
    raise NotImplementedError("write your pallas kernel here")


# speedup vs baseline: 2.4450x; 2.4450x over previous
import functools
import math

import jax
import jax.numpy as jnp
from jax import lax
from jax.experimental import pallas as pl
from jax.experimental.pallas import tpu as pltpu
from jax.experimental.pallas import tpu_sc as plsc

N = 10000
E = 320000
IN_DIM = 128
HID = 128
NUM_HEADS = 2
HEAD_DIM = HID // NUM_HEADS
SIGMA = 0.5
THRESH = 0.2

NC = 2
NS = 16
L = 16
NW = NC * NS

EPT = E // NW
C = 64
NCHUNK = EPT // C
TAIL = EPT - NCHUNK * C
GROUPS = C // L
RPT = 632

SRC_W = 3 * HID
DST_W = 2 * HID


_BR = 1000


def _prep0_body(x_ref, w_ref, b_ref, g_ref, bb_ref, o_ref):
    h = jnp.dot(x_ref[...], w_ref[...], preferred_element_type=jnp.float32)
    h = jnp.maximum(h + b_ref[...], 0.0)
    mu = jnp.mean(h, axis=-1, keepdims=True)
    var = jnp.mean((h - mu) * (h - mu), axis=-1, keepdims=True)
    o_ref[...] = (h - mu) * lax.rsqrt(var + 1e-5) * g_ref[...] + bb_ref[...]


def _prep0(x, w0t, b0, ln_g, ln_b):
    n = x.shape[0]
    grid = (n // _BR,)
    return pl.pallas_call(
        _prep0_body,
        grid=grid,
        in_specs=[
            pl.BlockSpec((_BR, IN_DIM), lambda i: (i, 0)),
            pl.BlockSpec((IN_DIM, HID), lambda i: (0, 0)),
            pl.BlockSpec((1, HID), lambda i: (0, 0)),
            pl.BlockSpec((1, HID), lambda i: (0, 0)),
            pl.BlockSpec((1, HID), lambda i: (0, 0)),
        ],
        out_specs=pl.BlockSpec((_BR, HID), lambda i: (i, 0)),
        out_shape=jax.ShapeDtypeStruct((n, HID), jnp.float32),
    )(x, w0t, b0, ln_g, ln_b)


def _prep_layer_body(h_ref, wq_ref, bq_ref, wk_ref, bk_ref, wv_ref, bv_ref,
                     s_ref, d_ref):
    h = h_ref[...]
    q = jnp.dot(h, wq_ref[...], preferred_element_type=jnp.float32) + bq_ref[...]
    k = jnp.dot(h, wk_ref[...], preferred_element_type=jnp.float32) + bk_ref[...]
    v = jnp.dot(h, wv_ref[...], preferred_element_type=jnp.float32) + bv_ref[...]
    nr = jnp.sqrt(jnp.sum(h * h, axis=-1, keepdims=True))
    hn = h / jnp.maximum(nr, 1e-8)
    s_ref[:, 0:HID] = q
    s_ref[:, HID:2 * HID] = hn
    s_ref[:, 2 * HID:3 * HID] = v
    d_ref[:, 0:HID] = k
    d_ref[:, HID:2 * HID] = hn


def _prep_layer(h, wqt, bq, wkt, bk, wvt, bv):
    n = h.shape[0]
    grid = (n // _BR,)
    return pl.pallas_call(
        _prep_layer_body,
        grid=grid,
        in_specs=[
            pl.BlockSpec((_BR, HID), lambda i: (i, 0)),
            pl.BlockSpec((HID, HID), lambda i: (0, 0)),
            pl.BlockSpec((1, HID), lambda i: (0, 0)),
            pl.BlockSpec((HID, HID), lambda i: (0, 0)),
            pl.BlockSpec((1, HID), lambda i: (0, 0)),
            pl.BlockSpec((HID, HID), lambda i: (0, 0)),
            pl.BlockSpec((1, HID), lambda i: (0, 0)),
        ],
        out_specs=[
            pl.BlockSpec((_BR, SRC_W), lambda i: (i, 0)),
            pl.BlockSpec((_BR, DST_W), lambda i: (i, 0)),
        ],
        out_shape=[
            jax.ShapeDtypeStruct((n, SRC_W), jnp.float32),
            jax.ShapeDtypeStruct((n, DST_W), jnp.float32),
        ],
    )(h, wqt, bq, wkt, bk, wvt, bv)


def _add2_body(a_ref, b_ref, o_ref):
    o_ref[...] = a_ref[...] + b_ref[...]


def _add2(a, b):
    n = a.shape[0]
    grid = (n // _BR,)
    return pl.pallas_call(
        _add2_body,
        grid=grid,
        in_specs=[
            pl.BlockSpec((_BR, HID), lambda i: (i, 0)),
            pl.BlockSpec((_BR, HID), lambda i: (i, 0)),
        ],
        out_specs=pl.BlockSpec((_BR, HID), lambda i: (i, 0)),
        out_shape=jax.ShapeDtypeStruct((n, HID), jnp.float32),
    )(a, b)


def _dist_groups(sprow, dprow, dv, ngroups):
    lanes = lax.iota(jnp.int32, L)

    def group(gi, c2):
        eb = gi * L
        d2 = jnp.zeros((L,), jnp.float32)
        for j in range(L):
            e = eb + j
            df = sprow[e, pl.ds(0, L)] - dprow[e, pl.ds(0, L)]
            d2 = jnp.where(lanes == j, jnp.sum(df * df), d2)
        dv[pl.ds(eb, L)] = jnp.exp(-2.0 * d2)
        return c2

    lax.fori_loop(0, ngroups, group, 0)


def _dist_body(pos_hbm, sidx_hbm, tidx_hbm, out_hbm,
               sidx_v, tidx_v, dv, sprow, dprow, sidx_t, tidx_t, sem1, sem2):
    cid = lax.axis_index("c")
    sid = lax.axis_index("s")
    wid = cid * NS + sid
    base0 = wid * EPT

    def chunk(ci, carry):
        base = base0 + ci * C
        pltpu.sync_copy(sidx_hbm.at[pl.ds(base, C)], sidx_v)
        pltpu.sync_copy(tidx_hbm.at[pl.ds(base, C)], tidx_v)
        cp1 = pltpu.async_copy(pos_hbm.at[sidx_v], sprow, sem1)
        cp2 = pltpu.async_copy(pos_hbm.at[tidx_v], dprow, sem2)
        cp1.wait()
        cp2.wait()
        _dist_groups(sprow, dprow, dv, GROUPS)
        pltpu.sync_copy(dv, out_hbm.at[pl.ds(base, C)])
        return carry

    lax.fori_loop(0, NCHUNK, chunk, 0)
    base = base0 + NCHUNK * C
    pltpu.sync_copy(sidx_hbm.at[pl.ds(base, TAIL)], sidx_t)
    pltpu.sync_copy(tidx_hbm.at[pl.ds(base, TAIL)], tidx_t)
    cp1 = pltpu.async_copy(pos_hbm.at[sidx_t], sprow.at[pl.ds(0, TAIL)], sem1)
    cp2 = pltpu.async_copy(pos_hbm.at[tidx_t], dprow.at[pl.ds(0, TAIL)], sem2)
    cp1.wait()
    cp2.wait()
    _dist_groups(sprow, dprow, dv, TAIL // L)
    pltpu.sync_copy(dv.at[pl.ds(0, TAIL)], out_hbm.at[pl.ds(base, TAIL)])


def _dist_sc(pos_pad, sidx, tidx):
    mesh = plsc.VectorSubcoreMesh(core_axis_name="c", subcore_axis_name="s",
                                  num_cores=NC, num_subcores=NS)
    return pl.kernel(
        _dist_body,
        out_type=jax.ShapeDtypeStruct((E,), jnp.float32),
        mesh=mesh,
        compiler_params=pltpu.CompilerParams(needs_layout_passes=False),
        scratch_types=[
            pltpu.VMEM((C,), jnp.int32),
            pltpu.VMEM((C,), jnp.int32),
            pltpu.VMEM((C,), jnp.float32),
            pltpu.VMEM((C, HID), jnp.float32),
            pltpu.VMEM((C, HID), jnp.float32),
            pltpu.VMEM((TAIL,), jnp.int32),
            pltpu.VMEM((TAIL,), jnp.int32),
            pltpu.SemaphoreType.DMA,
            pltpu.SemaphoreType.DMA,
        ],
    )(pos_pad, sidx, tidx)


_INV_SQRT_HD = 1.0 / math.sqrt(HEAD_DIM)


def _edge_body(srcT, dstT, sidx_hbm, tidx_hbm, dist_hbm, g0_hbm, g1_hbm,
               zeros_hbm, out_hbm,
               sidx_v, tidx_v, dist_v, srows, drows, msg,
               sidx_t, tidx_t, g0v, g1v, acc, sem1, sem2):
    cid = lax.axis_index("c")
    sid = lax.axis_index("s")
    wid = cid * NS + sid
    zstart = jnp.minimum(sid * RPT, N - RPT)
    pltpu.sync_copy(zeros_hbm.at[pl.ds(zstart, RPT)],
                    acc.at[pl.ds(zstart, RPT)])
    pltpu.sync_copy(g0_hbm, g0v)
    pltpu.sync_copy(g1_hbm, g1v)
    w0 = 1.0 / (1.0 + jnp.exp(-g0v[...]))
    w1 = 1.0 / (1.0 + jnp.exp(-g1v[...]))
    plsc.subcore_barrier()
    base0 = wid * EPT

    def run_groups(ngroups):
        lanes = lax.iota(jnp.int32, L)

        def group(gi, c2):
            eb = gi * L

            def dots(j, fv):
                f0, f1, gs = fv
                e = eb + j
                a0 = srows[e, pl.ds(0, L)] * drows[e, pl.ds(0, L)]
                a0 += srows[e, pl.ds(16, L)] * drows[e, pl.ds(16, L)]
                a0 += srows[e, pl.ds(32, L)] * drows[e, pl.ds(32, L)]
                a0 += srows[e, pl.ds(48, L)] * drows[e, pl.ds(48, L)]
                a1 = srows[e, pl.ds(64, L)] * drows[e, pl.ds(64, L)]
                a1 += srows[e, pl.ds(80, L)] * drows[e, pl.ds(80, L)]
                a1 += srows[e, pl.ds(96, L)] * drows[e, pl.ds(96, L)]
                a1 += srows[e, pl.ds(112, L)] * drows[e, pl.ds(112, L)]
                g = srows[e, pl.ds(128, L)] * drows[e, pl.ds(128, L)]
                for t in range(1, 8):
                    g += (srows[e, pl.ds(128 + t * L, L)]
                          * drows[e, pl.ds(128 + t * L, L)])
                lane = lanes == j
                return (jnp.where(lane, jnp.sum(a0), f0),
                        jnp.where(lane, jnp.sum(a1), f1),
                        jnp.where(lane, jnp.sum(g), gs))

            zero = jnp.zeros((L,), jnp.float32)
            f0, f1, gs = lax.fori_loop(0, L, dots, (zero, zero, zero))
            f0 = f0 * _INV_SQRT_HD
            f1 = f1 * _INV_SQRT_HD
            dd = dist_v[pl.ds(eb, L)]
            r0 = w0 * gs + (1.0 - w0) * (f0 * dd)
            r1 = w1 * gs + (1.0 - w1) * (f1 * dd)
            at0 = jnp.where(r0 >= 0.0, r0, 0.2 * r0)
            at1 = jnp.where(r1 >= 0.0, r1, 0.2 * r1)
            keep = (at0 + at1) * 0.5 > THRESH
            s0 = jnp.where(keep, at0, 0.0)
            s1 = jnp.where(keep, at1, 0.0)
            for j in range(L):
                e = eb + j
                c0 = jnp.full((L,), s0[j])
                c1 = jnp.full((L,), s1[j])
                for t in range(4):
                    msg[e, pl.ds(t * L, L)] = (
                        srows[e, pl.ds(2 * HID + t * L, L)] * c0)
                for t in range(4):
                    msg[e, pl.ds(64 + t * L, L)] = (
                        srows[e, pl.ds(2 * HID + 64 + t * L, L)] * c1)
            return c2

        lax.fori_loop(0, ngroups, group, 0)

    def chunk(ci, carry):
        base = base0 + ci * C
        pltpu.sync_copy(sidx_hbm.at[pl.ds(base, C)], sidx_v)
        pltpu.sync_copy(tidx_hbm.at[pl.ds(base, C)], tidx_v)
        pltpu.sync_copy(dist_hbm.at[pl.ds(base, C)], dist_v)
        cp1 = pltpu.async_copy(srcT.at[sidx_v], srows, sem1)
        cp2 = pltpu.async_copy(dstT.at[tidx_v], drows, sem2)
        cp1.wait()
        cp2.wait()
        run_groups(GROUPS)
        pltpu.sync_copy(msg, acc.at[tidx_v], add=True)
        return carry

    lax.fori_loop(0, NCHUNK, chunk, 0)
    base = base0 + NCHUNK * C
    pltpu.sync_copy(sidx_hbm.at[pl.ds(base, TAIL)], sidx_t)
    pltpu.sync_copy(tidx_hbm.at[pl.ds(base, TAIL)], tidx_t)
    pltpu.sync_copy(dist_hbm.at[pl.ds(base, TAIL)], dist_v.at[pl.ds(0, TAIL)])
    cp1 = pltpu.async_copy(srcT.at[sidx_t], srows.at[pl.ds(0, TAIL)], sem1)
    cp2 = pltpu.async_copy(dstT.at[tidx_t], drows.at[pl.ds(0, TAIL)], sem2)
    cp1.wait()
    cp2.wait()
    run_groups(TAIL // L)
    pltpu.sync_copy(msg.at[pl.ds(0, TAIL)], acc.at[tidx_t], add=True)
    plsc.subcore_barrier()
    rstart = jnp.minimum(sid * RPT, N - RPT)
    pltpu.sync_copy(acc.at[pl.ds(rstart, RPT)],
                    out_hbm.at[cid, pl.ds(rstart, RPT)])


def _edge_sc(srcT, dstT, sidx, tidx, dist, g0, g1, zeros):
    mesh = plsc.VectorSubcoreMesh(core_axis_name="c", subcore_axis_name="s",
                                  num_cores=NC, num_subcores=NS)
    return pl.kernel(
        _edge_body,
        out_type=jax.ShapeDtypeStruct((NC, N, HID), jnp.float32),
        mesh=mesh,
        compiler_params=pltpu.CompilerParams(needs_layout_passes=False),
        scratch_types=[
            pltpu.VMEM((C,), jnp.int32),
            pltpu.VMEM((C,), jnp.int32),
            pltpu.VMEM((C,), jnp.float32),
            pltpu.VMEM((C, SRC_W), jnp.float32),
            pltpu.VMEM((C, DST_W), jnp.float32),
            pltpu.VMEM((C, HID), jnp.float32),
            pltpu.VMEM((TAIL,), jnp.int32),
            pltpu.VMEM((TAIL,), jnp.int32),
            pltpu.VMEM((L,), jnp.float32),
            pltpu.VMEM((L,), jnp.float32),
            pltpu.VMEM_SHARED((N, HID), jnp.float32),
            pltpu.SemaphoreType.DMA,
            pltpu.SemaphoreType.DMA,
        ],
    )(srcT, dstT, sidx, tidx, dist, g0, g1, zeros)


def kernel(x, edge_index, positions, W0, b0, ln_g, ln_b,
           Wq, bq, Wk, bk, Wv, bv, gate):
    sidx = edge_index[0]
    tidx = edge_index[1]
    pos_pad = jnp.pad(positions, ((0, 0), (0, HID - 2)))
    h = _prep0(x, W0.T, b0.reshape(1, HID),
               ln_g.reshape(1, HID), ln_b.reshape(1, HID))
    dist = _dist_sc(pos_pad, sidx, tidx)
    zeros = jnp.zeros((N, HID), jnp.float32)
    for i in range(3):
        srcT, dstT = _prep_layer(h, Wq[i].T, bq[i].reshape(1, HID),
                                 Wk[i].T, bk[i].reshape(1, HID),
                                 Wv[i].T, bv[i].reshape(1, HID))
        g0 = jnp.full((L,), gate[i, 0], jnp.float32)
        g1 = jnp.full((L,), gate[i, 1], jnp.float32)
        parts = _edge_sc(srcT, dstT, sidx, tidx, dist, g0, g1, zeros)
        h = _add2(parts[0], parts[1])
    return h


# speedup vs baseline: 6.5908x; 2.6957x over previous
import functools
import math

import jax
import jax.numpy as jnp
from jax import lax
from jax.experimental import pallas as pl
from jax.experimental.pallas import tpu as pltpu
from jax.experimental.pallas import tpu_sc as plsc

N = 10000
E = 320000
IN_DIM = 128
HID = 128
NUM_HEADS = 2
HEAD_DIM = HID // NUM_HEADS
SIGMA = 0.5
THRESH = 0.2

NC = 2
NS = 16
L = 16
NW = NC * NS

EPT = E // NW
C = 32
RPT = 632

SRC_W = 3 * HID
DST_W = 2 * HID


_BR = 1000


def _prep0_body(x_ref, w_ref, b_ref, g_ref, bb_ref, o_ref):
    h = jnp.dot(x_ref[...], w_ref[...], preferred_element_type=jnp.float32)
    h = jnp.maximum(h + b_ref[...], 0.0)
    mu = jnp.mean(h, axis=-1, keepdims=True)
    var = jnp.mean((h - mu) * (h - mu), axis=-1, keepdims=True)
    o_ref[...] = (h - mu) * lax.rsqrt(var + 1e-5) * g_ref[...] + bb_ref[...]


def _prep0(x, w0t, b0, ln_g, ln_b):
    n = x.shape[0]
    grid = (n // _BR,)
    return pl.pallas_call(
        _prep0_body,
        grid=grid,
        in_specs=[
            pl.BlockSpec((_BR, IN_DIM), lambda i: (i, 0)),
            pl.BlockSpec((IN_DIM, HID), lambda i: (0, 0)),
            pl.BlockSpec((1, HID), lambda i: (0, 0)),
            pl.BlockSpec((1, HID), lambda i: (0, 0)),
            pl.BlockSpec((1, HID), lambda i: (0, 0)),
        ],
        out_specs=pl.BlockSpec((_BR, HID), lambda i: (i, 0)),
        out_shape=jax.ShapeDtypeStruct((n, HID), jnp.float32),
    )(x, w0t, b0, ln_g, ln_b)


def _prep_layer_body(h_ref, wq_ref, bq_ref, wk_ref, bk_ref, wv_ref, bv_ref,
                     s_ref, d_ref):
    h = h_ref[...]
    q = jnp.dot(h, wq_ref[...], preferred_element_type=jnp.float32) + bq_ref[...]
    k = jnp.dot(h, wk_ref[...], preferred_element_type=jnp.float32) + bk_ref[...]
    v = jnp.dot(h, wv_ref[...], preferred_element_type=jnp.float32) + bv_ref[...]
    nr = jnp.sqrt(jnp.sum(h * h, axis=-1, keepdims=True))
    hn = h / jnp.maximum(nr, 1e-8)
    s_ref[:, 0:HID] = q
    s_ref[:, HID:2 * HID] = hn
    s_ref[:, 2 * HID:3 * HID] = v
    d_ref[:, 0:HID] = k
    d_ref[:, HID:2 * HID] = hn


def _prep_layer(h, wqt, bq, wkt, bk, wvt, bv):
    n = h.shape[0]
    grid = (n // _BR,)
    return pl.pallas_call(
        _prep_layer_body,
        grid=grid,
        in_specs=[
            pl.BlockSpec((_BR, HID), lambda i: (i, 0)),
            pl.BlockSpec((HID, HID), lambda i: (0, 0)),
            pl.BlockSpec((1, HID), lambda i: (0, 0)),
            pl.BlockSpec((HID, HID), lambda i: (0, 0)),
            pl.BlockSpec((1, HID), lambda i: (0, 0)),
            pl.BlockSpec((HID, HID), lambda i: (0, 0)),
            pl.BlockSpec((1, HID), lambda i: (0, 0)),
        ],
        out_specs=[
            pl.BlockSpec((_BR, SRC_W), lambda i: (i, 0)),
            pl.BlockSpec((_BR, DST_W), lambda i: (i, 0)),
        ],
        out_shape=[
            jax.ShapeDtypeStruct((n, SRC_W), jnp.float32),
            jax.ShapeDtypeStruct((n, DST_W), jnp.float32),
        ],
    )(h, wqt, bq, wkt, bk, wvt, bv)


def _add2_body(a_ref, b_ref, o_ref):
    o_ref[...] = a_ref[...] + b_ref[...]


def _add2(a, b):
    n = a.shape[0]
    grid = (n // _BR,)
    return pl.pallas_call(
        _add2_body,
        grid=grid,
        in_specs=[
            pl.BlockSpec((_BR, HID), lambda i: (i, 0)),
            pl.BlockSpec((_BR, HID), lambda i: (i, 0)),
        ],
        out_specs=pl.BlockSpec((_BR, HID), lambda i: (i, 0)),
        out_shape=jax.ShapeDtypeStruct((n, HID), jnp.float32),
    )(a, b)


_CD = 512
_NCHD = EPT // _CD
_TAILD = EPT - _NCHD * _CD


def _dist_body(px_hbm, py_hbm, sidx_hbm, tidx_hbm, out_hbm,
               pxv, pyv, sidx_v, tidx_v, dv):
    cid = lax.axis_index("c")
    sid = lax.axis_index("s")
    wid = cid * NS + sid
    pltpu.sync_copy(px_hbm, pxv)
    pltpu.sync_copy(py_hbm, pyv)
    base0 = wid * EPT

    def groups(ngroups):
        def group(gi, c2):
            off = gi * L
            sv = sidx_v[pl.ds(off, L)]
            tv = tidx_v[pl.ds(off, L)]
            xs = plsc.load_gather(pxv, [sv])
            xt = plsc.load_gather(pxv, [tv])
            ys = plsc.load_gather(pyv, [sv])
            yt = plsc.load_gather(pyv, [tv])
            dx = xs - xt
            dy = ys - yt
            dv[pl.ds(off, L)] = jnp.exp(-2.0 * (dx * dx + dy * dy))
            return c2

        lax.fori_loop(0, ngroups, group, 0)

    def chunk(ci, carry):
        base = base0 + ci * _CD
        pltpu.sync_copy(sidx_hbm.at[pl.ds(base, _CD)], sidx_v)
        pltpu.sync_copy(tidx_hbm.at[pl.ds(base, _CD)], tidx_v)
        groups(_CD // L)
        pltpu.sync_copy(dv, out_hbm.at[pl.ds(base, _CD)])
        return carry

    lax.fori_loop(0, _NCHD, chunk, 0)
    base = base0 + _NCHD * _CD
    pltpu.sync_copy(sidx_hbm.at[pl.ds(base, _TAILD)], sidx_v.at[pl.ds(0, _TAILD)])
    pltpu.sync_copy(tidx_hbm.at[pl.ds(base, _TAILD)], tidx_v.at[pl.ds(0, _TAILD)])
    groups(_TAILD // L)
    pltpu.sync_copy(dv.at[pl.ds(0, _TAILD)], out_hbm.at[pl.ds(base, _TAILD)])


def _dist_sc(px, py, sidx, tidx):
    mesh = plsc.VectorSubcoreMesh(core_axis_name="c", subcore_axis_name="s",
                                  num_cores=NC, num_subcores=NS)
    return pl.kernel(
        _dist_body,
        out_type=jax.ShapeDtypeStruct((E,), jnp.float32),
        mesh=mesh,
        compiler_params=pltpu.CompilerParams(needs_layout_passes=False),
        scratch_types=[
            pltpu.VMEM((N,), jnp.float32),
            pltpu.VMEM((N,), jnp.float32),
            pltpu.VMEM((_CD,), jnp.int32),
            pltpu.VMEM((_CD,), jnp.int32),
            pltpu.VMEM((_CD,), jnp.float32),
        ],
    )(px, py, sidx, tidx)


_INV_SQRT_HD = 1.0 / math.sqrt(HEAD_DIM)


CS = 384
NSUP = EPT // CS
CPS = CS // C


def _edge_body(srcT, dstT, sidx_hbm, tidx_hbm, dist_hbm, g0_hbm, g1_hbm,
               zeros_hbm, out_hbm,
               stg0_s, stg0_t, stg0_d, stg1_s, stg1_t, stg1_d,
               stgT_s, stgT_t, stgT_d,
               srA, drA, srB, drB, msg, g0v, g1v, acc,
               sst0, sst1, sstT, sgA, sgB, ssc):
    cid = lax.axis_index("c")
    sid = lax.axis_index("s")
    wid = cid * NS + sid
    base0 = wid * EPT
    lanes = lax.iota(jnp.int32, L)

    def stage(stg_s, stg_t, stg_d, sem, base):
        pltpu.async_copy(sidx_hbm.at[pl.ds(base, CS)], stg_s, sem)
        pltpu.async_copy(tidx_hbm.at[pl.ds(base, CS)], stg_t, sem)
        pltpu.async_copy(dist_hbm.at[pl.ds(base, CS)], stg_d, sem)

    def wait_stage(stg_s, stg_t, stg_d, sem):
        pltpu.make_async_copy(sidx_hbm.at[pl.ds(0, CS)], stg_s, sem).wait()
        pltpu.make_async_copy(tidx_hbm.at[pl.ds(0, CS)], stg_t, sem).wait()
        pltpu.make_async_copy(dist_hbm.at[pl.ds(0, CS)], stg_d, sem).wait()

    stage(stg0_s, stg0_t, stg0_d, sst0, base0)
    stage(stg1_s, stg1_t, stg1_d, sst1, base0 + CS)
    pltpu.async_copy(sidx_hbm.at[pl.ds(base0 + NSUP * CS, L)], stgT_s, sstT)
    pltpu.async_copy(tidx_hbm.at[pl.ds(base0 + NSUP * CS, L)], stgT_t, sstT)
    pltpu.async_copy(dist_hbm.at[pl.ds(base0 + NSUP * CS, L)], stgT_d, sstT)
    zstart = jnp.minimum(sid * RPT, N - RPT)
    pltpu.sync_copy(zeros_hbm.at[pl.ds(zstart, RPT)],
                    acc.at[pl.ds(zstart, RPT)])
    pltpu.sync_copy(g0_hbm, g0v)
    pltpu.sync_copy(g1_hbm, g1v)
    w0 = 1.0 / (1.0 + jnp.exp(-g0v[...]))
    w1 = 1.0 / (1.0 + jnp.exp(-g1v[...]))
    zv = jnp.zeros((L,), jnp.float32)
    for j in range(C):
        for t in range(8):
            msg[j, pl.ds(t * L, L)] = zv
    plsc.subcore_barrier()
    pltpu.async_copy(msg.at[pl.ds(0, L)], acc.at[lanes], ssc, add=True)
    pltpu.async_copy(msg.at[pl.ds(L, L)], acc.at[lanes], ssc, add=True)

    def drain_scatter(k):
        for _ in range(k):
            pltpu.make_async_copy(msg.at[pl.ds(0, L)], acc.at[lanes],
                                  ssc).wait()

    def fire_gather(stg_s, stg_t, srX, drX, sgX, c):
        off = pl.multiple_of(c * C, C)
        pltpu.async_copy(srcT.at[stg_s.at[pl.ds(off, C)]], srX, sgX)
        pltpu.async_copy(dstT.at[stg_t.at[pl.ds(off, C)]], drX, sgX)

    def wait_gather(stg_s, stg_t, srX, drX, sgX):
        pltpu.make_async_copy(srcT.at[stg_s.at[pl.ds(0, C)]], srX, sgX).wait()
        pltpu.make_async_copy(dstT.at[stg_t.at[pl.ds(0, C)]], drX, sgX).wait()

    def compute_group(srX, drX, stg_t, stg_d, soff, grow, ndrain):
        def dots(j, fv):
            f0, f1, gs = fv
            e = grow + j
            a0 = srX[e, pl.ds(0, L)] * drX[e, pl.ds(0, L)]
            a0 += srX[e, pl.ds(16, L)] * drX[e, pl.ds(16, L)]
            a0 += srX[e, pl.ds(32, L)] * drX[e, pl.ds(32, L)]
            a0 += srX[e, pl.ds(48, L)] * drX[e, pl.ds(48, L)]
            a1 = srX[e, pl.ds(64, L)] * drX[e, pl.ds(64, L)]
            a1 += srX[e, pl.ds(80, L)] * drX[e, pl.ds(80, L)]
            a1 += srX[e, pl.ds(96, L)] * drX[e, pl.ds(96, L)]
            a1 += srX[e, pl.ds(112, L)] * drX[e, pl.ds(112, L)]
            g = srX[e, pl.ds(128, L)] * drX[e, pl.ds(128, L)]
            for t in range(1, 8):
                g += (srX[e, pl.ds(128 + t * L, L)]
                      * drX[e, pl.ds(128 + t * L, L)])
            lane = lanes == j
            return (jnp.where(lane, jnp.sum(a0), f0),
                    jnp.where(lane, jnp.sum(a1), f1),
                    jnp.where(lane, jnp.sum(g), gs))

        f0, f1, gs = lax.fori_loop(0, L, dots, (zv, zv, zv))
        f0 = f0 * _INV_SQRT_HD
        f1 = f1 * _INV_SQRT_HD
        dd = stg_d[pl.ds(soff, L)]
        r0 = w0 * gs + (1.0 - w0) * (f0 * dd)
        r1 = w1 * gs + (1.0 - w1) * (f1 * dd)
        at0 = jnp.where(r0 >= 0.0, r0, 0.2 * r0)
        at1 = jnp.where(r1 >= 0.0, r1, 0.2 * r1)
        keep = (at0 + at1) * 0.5 > THRESH
        s0 = jnp.where(keep, at0, 0.0)
        s1 = jnp.where(keep, at1, 0.0)
        drain_scatter(ndrain)
        for j in range(L):
            e = grow + j
            c0 = jnp.full((L,), s0[j])
            c1 = jnp.full((L,), s1[j])
            for t in range(4):
                msg[e, pl.ds(t * L, L)] = (
                    srX[e, pl.ds(2 * HID + t * L, L)] * c0)
            for t in range(4):
                msg[e, pl.ds(64 + t * L, L)] = (
                    srX[e, pl.ds(2 * HID + 64 + t * L, L)] * c1)
        tvec = stg_t[pl.ds(soff, L)]
        pltpu.async_copy(msg.at[pl.ds(grow, L)], acc.at[tvec], ssc, add=True)

    def compute_chunk(srX, drX, stg_t, stg_d, c):
        soff = c * C
        compute_group(srX, drX, stg_t, stg_d, soff, 0, 2)
        compute_group(srX, drX, stg_t, stg_d, soff + L, L, 0)

    def do_super(stg_s, stg_t, stg_d, sem_st, sb, fire_next):
        wait_stage(stg_s, stg_t, stg_d, sem_st)
        fire_gather(stg_s, stg_t, srA, drA, sgA, 0)

        def pair(j, carry):
            fire_gather(stg_s, stg_t, srB, drB, sgB, 2 * j + 1)
            wait_gather(stg_s, stg_t, srA, drA, sgA)
            compute_chunk(srA, drA, stg_t, stg_d, 2 * j)

            @pl.when(j < CPS // 2 - 1)
            def _():
                fire_gather(stg_s, stg_t, srA, drA, sgA, 2 * j + 2)

            wait_gather(stg_s, stg_t, srB, drB, sgB)
            compute_chunk(srB, drB, stg_t, stg_d, 2 * j + 1)
            return carry

        lax.fori_loop(0, CPS // 2, pair, 0)

        @pl.when(fire_next)
        def _():
            stage(stg_s, stg_t, stg_d, sem_st, sb + 2 * CS)

    def pairsup(k, carry):
        sb = base0 + 2 * k * CS
        do_super(stg0_s, stg0_t, stg0_d, sst0, sb, k < NSUP // 2 - 1)
        do_super(stg1_s, stg1_t, stg1_d, sst1, sb + CS, k < NSUP // 2 - 1)
        return carry

    lax.fori_loop(0, NSUP // 2, pairsup, 0)
    pltpu.make_async_copy(sidx_hbm.at[pl.ds(0, L)], stgT_s, sstT).wait()
    pltpu.make_async_copy(tidx_hbm.at[pl.ds(0, L)], stgT_t, sstT).wait()
    pltpu.make_async_copy(dist_hbm.at[pl.ds(0, L)], stgT_d, sstT).wait()
    pltpu.async_copy(srcT.at[stgT_s], srA.at[pl.ds(0, L)], sgA)
    pltpu.async_copy(dstT.at[stgT_t], drA.at[pl.ds(0, L)], sgA)
    pltpu.make_async_copy(srcT.at[stgT_s], srA.at[pl.ds(0, L)], sgA).wait()
    pltpu.make_async_copy(dstT.at[stgT_t], drA.at[pl.ds(0, L)], sgA).wait()
    compute_group(srA, drA, stgT_t, stgT_d, 0, 0, 2)
    drain_scatter(1)
    plsc.subcore_barrier()
    rstart = jnp.minimum(sid * RPT, N - RPT)
    pltpu.sync_copy(acc.at[pl.ds(rstart, RPT)],
                    out_hbm.at[cid, pl.ds(rstart, RPT)])


def _edge_sc(srcT, dstT, sidx, tidx, dist, g0, g1, zeros):
    mesh = plsc.VectorSubcoreMesh(core_axis_name="c", subcore_axis_name="s",
                                  num_cores=NC, num_subcores=NS)
    return pl.kernel(
        _edge_body,
        out_type=jax.ShapeDtypeStruct((NC, N, HID), jnp.float32),
        mesh=mesh,
        compiler_params=pltpu.CompilerParams(needs_layout_passes=False),
        scratch_types=[
            pltpu.VMEM((CS,), jnp.int32),
            pltpu.VMEM((CS,), jnp.int32),
            pltpu.VMEM((CS,), jnp.float32),
            pltpu.VMEM((CS,), jnp.int32),
            pltpu.VMEM((CS,), jnp.int32),
            pltpu.VMEM((CS,), jnp.float32),
            pltpu.VMEM((L,), jnp.int32),
            pltpu.VMEM((L,), jnp.int32),
            pltpu.VMEM((L,), jnp.float32),
            pltpu.VMEM((C, SRC_W), jnp.float32),
            pltpu.VMEM((C, DST_W), jnp.float32),
            pltpu.VMEM((C, SRC_W), jnp.float32),
            pltpu.VMEM((C, DST_W), jnp.float32),
            pltpu.VMEM((C, HID), jnp.float32),
            pltpu.VMEM((L,), jnp.float32),
            pltpu.VMEM((L,), jnp.float32),
            pltpu.VMEM_SHARED((N, HID), jnp.float32),
            pltpu.SemaphoreType.DMA,
            pltpu.SemaphoreType.DMA,
            pltpu.SemaphoreType.DMA,
            pltpu.SemaphoreType.DMA,
            pltpu.SemaphoreType.DMA,
            pltpu.SemaphoreType.DMA,
        ],
    )(srcT, dstT, sidx, tidx, dist, g0, g1, zeros)


def kernel(x, edge_index, positions, W0, b0, ln_g, ln_b,
           Wq, bq, Wk, bk, Wv, bv, gate):
    sidx = edge_index[0]
    tidx = edge_index[1]
    h = _prep0(x, W0.T, b0.reshape(1, HID),
               ln_g.reshape(1, HID), ln_b.reshape(1, HID))
    dist = _dist_sc(positions[:, 0], positions[:, 1], sidx, tidx)
    zeros = jnp.zeros((N, HID), jnp.float32)
    for i in range(3):
        srcT, dstT = _prep_layer(h, Wq[i].T, bq[i].reshape(1, HID),
                                 Wk[i].T, bk[i].reshape(1, HID),
                                 Wv[i].T, bv[i].reshape(1, HID))
        g0 = jnp.full((L,), gate[i, 0], jnp.float32)
        g1 = jnp.full((L,), gate[i, 1], jnp.float32)
        parts = _edge_sc(srcT, dstT, sidx, tidx, dist, g0, g1, zeros)
        h = _add2(parts[0], parts[1])
    return h
